# baseline (speedup 1.0000x reference)
import functools

import jax
import jax.numpy as jnp
from jax import lax
from jax.experimental import pallas as pl
from jax.experimental.pallas import tpu as pltpu
from jax.experimental.pallas import tpu_sc as plsc

T = 4096
HIDDEN = 2048
INTER = 1024
E = 8
K = 2
TK = T * K

BM = 256
NB = TK // BM + E
P = NB * BM

NC = 2
NS = 16
NW = NC * NS

GCH = 64
CT = 16


@functools.lru_cache(maxsize=None)
def _sc_mesh():
    return plsc.VectorSubcoreMesh(
        core_axis_name="c", subcore_axis_name="s",
        num_cores=NC, num_subcores=NS)


def _gather_body(tab_hbm, ptok_hbm, pos_hbm, out_hbm, gidx_v, sidx_v, rows_v,
                 sem_g, sem_s):
    wid = lax.axis_index("s") * NC + lax.axis_index("c")
    per_w = TK // NW
    base = wid * per_w
    for c in range(per_w // GCH):
        lo = base + c * GCH
        pltpu.sync_copy(ptok_hbm.at[pl.ds(lo, GCH)], gidx_v)
        pltpu.sync_copy(pos_hbm.at[pl.ds(lo, GCH)], sidx_v)
        pltpu.async_copy(tab_hbm.at[gidx_v], rows_v, sem_g).wait()
        pltpu.async_copy(rows_v, out_hbm.at[sidx_v], sem_s).wait()


@functools.lru_cache(maxsize=None)
def _gather_call():
    return pl.kernel(
        _gather_body,
        out_type=jax.ShapeDtypeStruct((P, HIDDEN // 2), jnp.int32),
        mesh=_sc_mesh(),
        scratch_types=[
            pltpu.VMEM((GCH,), jnp.int32),
            pltpu.VMEM((GCH,), jnp.int32),
            pltpu.VMEM((GCH, HIDDEN // 2), jnp.int32),
            pltpu.SemaphoreType.DMA,
            pltpu.SemaphoreType.DMA,
        ],
    )


def _ffn_body(be_ref, x_ref, wgu_ref, wd_ref, o_ref):
    x = x_ref[...]
    gu = jnp.dot(x, wgu_ref[0], preferred_element_type=jnp.float32)
    g = gu[:, :INTER]
    u = gu[:, INTER:]
    h = (g * lax.logistic(g) * u).astype(jnp.bfloat16)
    o_ref[...] = jnp.dot(h, wd_ref[0], preferred_element_type=jnp.float32)


def _grouped_ffn(bexp, xp, wgu, wd):
    grid_spec = pltpu.PrefetchScalarGridSpec(
        num_scalar_prefetch=1,
        grid=(NB,),
        in_specs=[
            pl.BlockSpec((BM, HIDDEN), lambda b, be: (b, 0)),
            pl.BlockSpec((1, HIDDEN, 2 * INTER), lambda b, be: (be[b], 0, 0)),
            pl.BlockSpec((1, INTER, HIDDEN), lambda b, be: (be[b], 0, 0)),
        ],
        out_specs=pl.BlockSpec((BM, HIDDEN), lambda b, be: (b, 0)),
    )
    return pl.pallas_call(
        _ffn_body,
        grid_spec=grid_spec,
        out_shape=jax.ShapeDtypeStruct((P, HIDDEN), jnp.float32),
        compiler_params=pltpu.CompilerParams(
            dimension_semantics=("arbitrary",),
            vmem_limit_bytes=100 * 1024 * 1024,
        ),
    )(bexp, xp, wgu, wd)


def _combine_body(po_hbm, pa_hbm, pb_hbm, wa_hbm, wb_hbm, out_hbm,
                  ia_v, ib_v, wa_v, wb_v, bufa, bufb, sa, sb):
    wid = lax.axis_index("s") * NC + lax.axis_index("c")
    per_w = T // NW
    tbase = wid * per_w
    for c in range(per_w // CT):
        lo = tbase + c * CT
        pltpu.sync_copy(pa_hbm.at[pl.ds(lo, CT)], ia_v)
        pltpu.sync_copy(pb_hbm.at[pl.ds(lo, CT)], ib_v)
        pltpu.sync_copy(wa_hbm.at[pl.ds(lo, CT)], wa_v)
        pltpu.sync_copy(wb_hbm.at[pl.ds(lo, CT)], wb_v)
        cpa = pltpu.async_copy(po_hbm.at[ia_v], bufa, sa)
        cpb = pltpu.async_copy(po_hbm.at[ib_v], bufb, sb)
        cpa.wait()
        cpb.wait()
        wav = wa_v[...]
        wbv = wb_v[...]
        for r in range(CT):
            wa = wav[r]
            wb = wbv[r]

            def dbody(d, _, r=r, wa=wa, wb=wb):
                s = pl.ds(d * 16, 16)
                bufa[r, s] = wa * bufa[r, s] + wb * bufb[r, s]
                return 0

            lax.fori_loop(0, HIDDEN // 16, dbody, 0)
        pltpu.sync_copy(bufa, out_hbm.at[pl.ds(lo, CT)])


@functools.lru_cache(maxsize=None)
def _combine_call():
    return pl.kernel(
        _combine_body,
        out_type=jax.ShapeDtypeStruct((T, HIDDEN), jnp.float32),
        mesh=_sc_mesh(),
        scratch_types=[
            pltpu.VMEM((CT,), jnp.int32),
            pltpu.VMEM((CT,), jnp.int32),
            pltpu.VMEM((CT,), jnp.float32),
            pltpu.VMEM((CT,), jnp.float32),
            pltpu.VMEM((CT, HIDDEN), jnp.float32),
            pltpu.VMEM((CT, HIDDEN), jnp.float32),
            pltpu.SemaphoreType.DMA,
            pltpu.SemaphoreType.DMA,
        ],
    )


def kernel(hidden_states, top_k_index, top_k_weights, gate_up_proj, down_proj):
    idx = top_k_index.astype(jnp.int32).reshape(-1)
    oh = (idx[:, None] == jnp.arange(E, dtype=jnp.int32)).astype(jnp.int32)
    csum = jnp.cumsum(oh, axis=0)
    counts = csum[-1]
    rank = jnp.take_along_axis(csum, idx[:, None], axis=1)[:, 0] - 1
    nb = (counts + BM - 1) // BM
    bcum = jnp.cumsum(nb)
    off = (bcum - nb) * BM
    pos = (off[idx] + rank).astype(jnp.int32)
    bexp = jnp.minimum(
        jnp.searchsorted(bcum, jnp.arange(NB, dtype=jnp.int32), side="right"),
        E - 1,
    ).astype(jnp.int32)

    pair_tok = jnp.arange(TK, dtype=jnp.int32) // K
    hb = hidden_states.astype(jnp.bfloat16)
    tab_i32 = lax.bitcast_convert_type(
        hb.reshape(T, HIDDEN // 2, 2), jnp.int32)

    packed_i32 = _gather_call()(tab_i32, pair_tok, pos)
    packed_bf16 = lax.bitcast_convert_type(
        packed_i32, jnp.bfloat16).reshape(P, HIDDEN)

    wgu = jnp.swapaxes(gate_up_proj, 1, 2).astype(jnp.bfloat16)
    wd = jnp.swapaxes(down_proj, 1, 2).astype(jnp.bfloat16)

    packed_out = _grouped_ffn(bexp, packed_bf16, wgu, wd)

    pos2 = pos.reshape(T, K)
    final = _combine_call()(
        packed_out,
        pos2[:, 0], pos2[:, 1],
        top_k_weights[:, 0].astype(jnp.float32),
        top_k_weights[:, 1].astype(jnp.float32),
    )
    return final


# speedup vs baseline: 2.9933x; 2.9933x over previous
import functools

import jax
import jax.numpy as jnp
from jax import lax
from jax.experimental import pallas as pl
from jax.experimental.pallas import tpu as pltpu
from jax.experimental.pallas import tpu_sc as plsc

T = 4096
HIDDEN = 2048
INTER = 1024
E = 8
K = 2
TK = T * K

BM = 256
NB = TK // BM + E
P = NB * BM

NC = 2
NS = 16
NW = NC * NS

GCH = 16
CT = 8


@functools.lru_cache(maxsize=None)
def _sc_mesh():
    return plsc.VectorSubcoreMesh(
        core_axis_name="c", subcore_axis_name="s",
        num_cores=NC, num_subcores=NS)


def _wid():
    return lax.axis_index("s") * NC + lax.axis_index("c")


def _gather_body(tab_hbm, ptok_hbm, pos_hbm, out_hbm,
                 gidx0, gidx1, sidx0, sidx1, rows0, rows1,
                 gsem0, gsem1, ssem0, ssem1):
    per_w = TK // NW
    base = _wid() * per_w
    n_ch = per_w // GCH
    gidx = (gidx0, gidx1)
    sidx = (sidx0, sidx1)
    rows = (rows0, rows1)
    gsem = (gsem0, gsem1)
    ssem = (ssem0, ssem1)

    def start(c):
        b = c % 2
        lo = base + c * GCH
        pltpu.sync_copy(ptok_hbm.at[pl.ds(lo, GCH)], gidx[b])
        pltpu.sync_copy(pos_hbm.at[pl.ds(lo, GCH)], sidx[b])
        return pltpu.async_copy(tab_hbm.at[gidx[b]], rows[b], gsem[b])

    gd = {0: start(0)}
    sd = {}
    for c in range(n_ch):
        b = c % 2
        if c + 1 < n_ch:
            if c - 1 in sd:
                sd[c - 1].wait()
            gd[c + 1] = start(c + 1)
        gd[c].wait()
        sd[c] = pltpu.async_copy(rows[b], out_hbm.at[sidx[b]], ssem[b])
    if n_ch - 2 >= 0 and n_ch - 2 in sd:
        sd[n_ch - 2].wait()
    sd[n_ch - 1].wait()


@functools.lru_cache(maxsize=None)
def _gather_call():
    return pl.kernel(
        _gather_body,
        out_type=jax.ShapeDtypeStruct((P, HIDDEN), jnp.float32),
        mesh=_sc_mesh(),
        scratch_types=[
            pltpu.VMEM((GCH,), jnp.int32),
            pltpu.VMEM((GCH,), jnp.int32),
            pltpu.VMEM((GCH,), jnp.int32),
            pltpu.VMEM((GCH,), jnp.int32),
            pltpu.VMEM((GCH, HIDDEN), jnp.float32),
            pltpu.VMEM((GCH, HIDDEN), jnp.float32),
            pltpu.SemaphoreType.DMA,
            pltpu.SemaphoreType.DMA,
            pltpu.SemaphoreType.DMA,
            pltpu.SemaphoreType.DMA,
        ],
    )


def _ffn_body(be_ref, x_ref, w_ref, wgu_ref, wd_ref, o_ref):
    x = x_ref[...].astype(jnp.bfloat16)
    gu = lax.dot_general(
        x, wgu_ref[0].astype(jnp.bfloat16),
        (((1,), (1,)), ((), ())),
        preferred_element_type=jnp.float32)
    g = gu[:, :INTER]
    u = gu[:, INTER:]
    h = (g * lax.logistic(g) * u).astype(jnp.bfloat16)
    out = lax.dot_general(
        h, wd_ref[0].astype(jnp.bfloat16),
        (((1,), (1,)), ((), ())),
        preferred_element_type=jnp.float32)
    o_ref[...] = out * w_ref[...]


def _grouped_ffn(bexp, xp, w_pos, wgu, wd):
    grid_spec = pltpu.PrefetchScalarGridSpec(
        num_scalar_prefetch=1,
        grid=(NB,),
        in_specs=[
            pl.BlockSpec((BM, HIDDEN), lambda b, be: (b, 0)),
            pl.BlockSpec((BM, 1), lambda b, be: (b, 0)),
            pl.BlockSpec((1, 2 * INTER, HIDDEN), lambda b, be: (be[b], 0, 0)),
            pl.BlockSpec((1, HIDDEN, INTER), lambda b, be: (be[b], 0, 0)),
        ],
        out_specs=pl.BlockSpec((BM, HIDDEN), lambda b, be: (b, 0)),
    )
    return pl.pallas_call(
        _ffn_body,
        grid_spec=grid_spec,
        out_shape=jax.ShapeDtypeStruct((P, HIDDEN), jnp.float32),
        compiler_params=pltpu.CompilerParams(
            dimension_semantics=("arbitrary",),
            vmem_limit_bytes=110 * 1024 * 1024,
        ),
    )(bexp, xp, w_pos, wgu, wd)


def _combine_body(po_hbm, pa_hbm, pb_hbm, out_hbm,
                  ia0, ia1, ib0, ib1, ba0, ba1, bb0, bb1,
                  sa0, sa1, sb0, sb1, so0, so1):
    per_w = T // NW
    tbase = _wid() * per_w
    n_ch = per_w // CT
    ia = (ia0, ia1)
    ib = (ib0, ib1)
    ba = (ba0, ba1)
    bb = (bb0, bb1)
    sa = (sa0, sa1)
    sb = (sb0, sb1)
    so = (so0, so1)

    def start(c):
        b = c % 2
        lo = tbase + c * CT
        pltpu.sync_copy(pa_hbm.at[pl.ds(lo, CT)], ia[b])
        pltpu.sync_copy(pb_hbm.at[pl.ds(lo, CT)], ib[b])
        return (pltpu.async_copy(po_hbm.at[ia[b]], ba[b], sa[b]),
                pltpu.async_copy(po_hbm.at[ib[b]], bb[b], sb[b]))

    gd = {0: start(0)}
    wd_ = {}
    for c in range(n_ch):
        b = c % 2
        if c + 1 < n_ch:
            if c - 1 in wd_:
                wd_[c - 1].wait()
            gd[c + 1] = start(c + 1)
        gd[c][0].wait()
        gd[c][1].wait()
        for r in range(CT):
            @plsc.parallel_loop(0, HIDDEN // 16, unroll=8)
            def dbody(d, r=r, b=b):
                s = pl.ds(d * 16, 16)
                ba[b][r, s] = ba[b][r, s] + bb[b][r, s]
        wd_[c] = pltpu.async_copy(
            ba[b], out_hbm.at[pl.ds(tbase + c * CT, CT)], so[b])
    if n_ch - 2 >= 0 and n_ch - 2 in wd_:
        wd_[n_ch - 2].wait()
    wd_[n_ch - 1].wait()


@functools.lru_cache(maxsize=None)
def _combine_call():
    return pl.kernel(
        _combine_body,
        out_type=jax.ShapeDtypeStruct((T, HIDDEN), jnp.float32),
        mesh=_sc_mesh(),
        scratch_types=[
            pltpu.VMEM((CT,), jnp.int32),
            pltpu.VMEM((CT,), jnp.int32),
            pltpu.VMEM((CT,), jnp.int32),
            pltpu.VMEM((CT,), jnp.int32),
            pltpu.VMEM((CT, HIDDEN), jnp.float32),
            pltpu.VMEM((CT, HIDDEN), jnp.float32),
            pltpu.VMEM((CT, HIDDEN), jnp.float32),
            pltpu.VMEM((CT, HIDDEN), jnp.float32),
            pltpu.SemaphoreType.DMA,
            pltpu.SemaphoreType.DMA,
            pltpu.SemaphoreType.DMA,
            pltpu.SemaphoreType.DMA,
            pltpu.SemaphoreType.DMA,
            pltpu.SemaphoreType.DMA,
        ],
    )


def kernel(hidden_states, top_k_index, top_k_weights, gate_up_proj, down_proj):
    idx = top_k_index.astype(jnp.int32).reshape(-1)
    oh = (idx[:, None] == jnp.arange(E, dtype=jnp.int32)).astype(jnp.int32)
    csum = jnp.cumsum(oh, axis=0)
    counts = csum[-1]
    rank = jnp.take_along_axis(csum, idx[:, None], axis=1)[:, 0] - 1
    nb = (counts + BM - 1) // BM
    bcum = jnp.cumsum(nb)
    off = (bcum - nb) * BM
    pos = (off[idx] + rank).astype(jnp.int32)
    bexp = jnp.minimum(
        jnp.searchsorted(bcum, jnp.arange(NB, dtype=jnp.int32), side="right"),
        E - 1,
    ).astype(jnp.int32)
    w_pos = jnp.zeros((P,), jnp.float32).at[pos].set(
        top_k_weights.reshape(-1).astype(jnp.float32)).reshape(P, 1)

    pair_tok = jnp.arange(TK, dtype=jnp.int32) // K

    packed_x = _gather_call()(hidden_states, pair_tok, pos)

    packed_out = _grouped_ffn(bexp, packed_x, w_pos, gate_up_proj, down_proj)

    pos2 = pos.reshape(T, K)
    final = _combine_call()(packed_out, pos2[:, 0], pos2[:, 1])
    return final


# speedup vs baseline: 3.1097x; 1.0389x over previous
import functools

import jax
import jax.numpy as jnp
from jax import lax
from jax.experimental import pallas as pl
from jax.experimental.pallas import tpu as pltpu
from jax.experimental.pallas import tpu_sc as plsc

T = 4096
HIDDEN = 2048
INTER = 1024
E = 8
K = 2
TK = T * K

BM = 256
NB = TK // BM + E
P = NB * BM

NC = 2
NS = 16
NW = NC * NS

GCH = 16
CT = 8


@functools.lru_cache(maxsize=None)
def _sc_mesh():
    return plsc.VectorSubcoreMesh(
        core_axis_name="c", subcore_axis_name="s",
        num_cores=NC, num_subcores=NS)


def _wid():
    return lax.axis_index("s") * NC + lax.axis_index("c")


def _gather_body(tab_hbm, pa_hbm, pb_hbm, out_hbm,
                 sidxa0, sidxa1, sidxb0, sidxb1, rows0, rows1,
                 gsem0, gsem1, ssema0, ssema1, ssemb0, ssemb1):
    per_w = T // NW
    base = _wid() * per_w
    n_ch = per_w // GCH
    sidxa = (sidxa0, sidxa1)
    sidxb = (sidxb0, sidxb1)
    rows = (rows0, rows1)
    gsem = (gsem0, gsem1)
    ssema = (ssema0, ssema1)
    ssemb = (ssemb0, ssemb1)

    def start(c):
        b = c % 2
        lo = base + c * GCH
        pltpu.sync_copy(pa_hbm.at[pl.ds(lo, GCH)], sidxa[b])
        pltpu.sync_copy(pb_hbm.at[pl.ds(lo, GCH)], sidxb[b])
        return pltpu.async_copy(tab_hbm.at[pl.ds(lo, GCH)], rows[b], gsem[b])

    gd = {0: start(0)}
    sd = {}
    for c in range(n_ch):
        b = c % 2
        if c + 1 < n_ch:
            if c - 1 in sd:
                sd[c - 1][0].wait()
                sd[c - 1][1].wait()
            gd[c + 1] = start(c + 1)
        gd[c].wait()
        sd[c] = (pltpu.async_copy(rows[b], out_hbm.at[sidxa[b]], ssema[b]),
                 pltpu.async_copy(rows[b], out_hbm.at[sidxb[b]], ssemb[b]))
    for c in (n_ch - 2, n_ch - 1):
        if c >= 0 and c in sd:
            sd[c][0].wait()
            sd[c][1].wait()


@functools.lru_cache(maxsize=None)
def _gather_call():
    return pl.kernel(
        _gather_body,
        out_type=jax.ShapeDtypeStruct((P, HIDDEN), jnp.float32),
        mesh=_sc_mesh(),
        scratch_types=[
            pltpu.VMEM((GCH,), jnp.int32),
            pltpu.VMEM((GCH,), jnp.int32),
            pltpu.VMEM((GCH,), jnp.int32),
            pltpu.VMEM((GCH,), jnp.int32),
            pltpu.VMEM((GCH, HIDDEN), jnp.float32),
            pltpu.VMEM((GCH, HIDDEN), jnp.float32),
            pltpu.SemaphoreType.DMA,
            pltpu.SemaphoreType.DMA,
            pltpu.SemaphoreType.DMA,
            pltpu.SemaphoreType.DMA,
            pltpu.SemaphoreType.DMA,
            pltpu.SemaphoreType.DMA,
        ],
    )


def _ffn_body(be_ref, x_ref, w_ref, wgu_ref, wd_ref, o_ref):
    @pl.when(pl.program_id(0) < be_ref[NB])
    def _():
        x = x_ref[...].astype(jnp.bfloat16)
        gu = lax.dot_general(
            x, wgu_ref[0].astype(jnp.bfloat16),
            (((1,), (1,)), ((), ())),
            preferred_element_type=jnp.float32)
        g = gu[:, :INTER]
        u = gu[:, INTER:]
        h = (g * lax.logistic(g) * u).astype(jnp.bfloat16)
        out = lax.dot_general(
            h, wd_ref[0].astype(jnp.bfloat16),
            (((1,), (1,)), ((), ())),
            preferred_element_type=jnp.float32)
        o_ref[...] = out * w_ref[...]


def _grouped_ffn(bexp, xp, w_pos, wgu, wd):
    grid_spec = pltpu.PrefetchScalarGridSpec(
        num_scalar_prefetch=1,
        grid=(NB,),
        in_specs=[
            pl.BlockSpec((BM, HIDDEN), lambda b, be: (b, 0)),
            pl.BlockSpec((BM, 1), lambda b, be: (b, 0)),
            pl.BlockSpec((1, 2 * INTER, HIDDEN), lambda b, be: (be[b], 0, 0)),
            pl.BlockSpec((1, HIDDEN, INTER), lambda b, be: (be[b], 0, 0)),
        ],
        out_specs=pl.BlockSpec((BM, HIDDEN), lambda b, be: (b, 0)),
    )
    return pl.pallas_call(
        _ffn_body,
        grid_spec=grid_spec,
        out_shape=jax.ShapeDtypeStruct((P, HIDDEN), jnp.float32),
        compiler_params=pltpu.CompilerParams(
            dimension_semantics=("arbitrary",),
            vmem_limit_bytes=110 * 1024 * 1024,
        ),
    )(bexp, xp, w_pos, wgu, wd)


def _combine_body(po_hbm, pos_hbm, out_hbm,
                  ii0, ii1, bi0, bi1, bo0, bo1,
                  sg0, sg1, so0, so1):
    per_w = T // NW
    tbase = _wid() * per_w
    n_ch = per_w // CT
    ii = (ii0, ii1)
    bi = (bi0, bi1)
    bo = (bo0, bo1)
    sg = (sg0, sg1)
    so = (so0, so1)

    def start(c):
        b = c % 2
        pltpu.sync_copy(pos_hbm.at[pl.ds((tbase + c * CT) * K, CT * K)], ii[b])
        return pltpu.async_copy(po_hbm.at[ii[b]], bi[b], sg[b])

    gd = {0: start(0)}
    wd_ = {}
    for c in range(n_ch):
        b = c % 2
        if c + 1 < n_ch:
            if c - 1 in wd_:
                wd_[c - 1].wait()
            gd[c + 1] = start(c + 1)
        gd[c].wait()
        for r in range(CT):
            @plsc.parallel_loop(0, HIDDEN // 16, unroll=8)
            def dbody(d, r=r, b=b):
                s = pl.ds(d * 16, 16)
                bo[b][r, s] = bi[b][2 * r, s] + bi[b][2 * r + 1, s]
        wd_[c] = pltpu.async_copy(
            bo[b], out_hbm.at[pl.ds(tbase + c * CT, CT)], so[b])
    for c in (n_ch - 2, n_ch - 1):
        if c >= 0 and c in wd_:
            wd_[c].wait()


@functools.lru_cache(maxsize=None)
def _combine_call():
    return pl.kernel(
        _combine_body,
        out_type=jax.ShapeDtypeStruct((T, HIDDEN), jnp.float32),
        mesh=_sc_mesh(),
        scratch_types=[
            pltpu.VMEM((CT * K,), jnp.int32),
            pltpu.VMEM((CT * K,), jnp.int32),
            pltpu.VMEM((CT * K, HIDDEN), jnp.float32),
            pltpu.VMEM((CT * K, HIDDEN), jnp.float32),
            pltpu.VMEM((CT, HIDDEN), jnp.float32),
            pltpu.VMEM((CT, HIDDEN), jnp.float32),
            pltpu.SemaphoreType.DMA,
            pltpu.SemaphoreType.DMA,
            pltpu.SemaphoreType.DMA,
            pltpu.SemaphoreType.DMA,
        ],
    )


def kernel(hidden_states, top_k_index, top_k_weights, gate_up_proj, down_proj):
    idx = top_k_index.astype(jnp.int32).reshape(-1)
    oh = (idx[:, None] == jnp.arange(E, dtype=jnp.int32)).astype(jnp.int32)
    csum = jnp.cumsum(oh, axis=0)
    counts = csum[-1]
    rank = jnp.take_along_axis(csum, idx[:, None], axis=1)[:, 0] - 1
    nb = (counts + BM - 1) // BM
    bcum = jnp.cumsum(nb)
    off = (bcum - nb) * BM
    pos = (off[idx] + rank).astype(jnp.int32)
    bexp = jnp.minimum(
        jnp.searchsorted(bcum, jnp.arange(NB, dtype=jnp.int32), side="right"),
        E - 1,
    ).astype(jnp.int32)
    bexp_ext = jnp.concatenate([bexp, bcum[-1:].astype(jnp.int32)])
    w_pos = jnp.zeros((P,), jnp.float32).at[pos].set(
        top_k_weights.reshape(-1).astype(jnp.float32)).reshape(P, 1)

    pos2 = pos.reshape(T, K)
    packed_x = _gather_call()(hidden_states, pos2[:, 0], pos2[:, 1])

    packed_out = _grouped_ffn(bexp_ext, packed_x, w_pos, gate_up_proj, down_proj)

    final = _combine_call()(packed_out, pos)
    return final
